# baseline (speedup 1.0000x reference)
import functools

import jax
import jax.numpy as jnp
from jax import lax
from jax.experimental import pallas as pl
from jax.experimental.pallas import tpu as pltpu
from jax.experimental.pallas import tpu_sc as plsc

N_NODES = 10000
NP = 10240
E = 320000
NC, NS, LANES = 2, 16, 16
NW = NC * NS
CHUNK = 128
TPW = 80
HTPW = TPW // 2
EP = NW * TPW * CHUNK
NFEAT = 128
NHID = 128
NCLASS = 16
ROWS_PT = NP // NS

_mesh = plsc.VectorSubcoreMesh(
    core_axis_name="c", subcore_axis_name="s", num_cores=NC, num_subcores=NS)


@functools.partial(
    pl.kernel,
    out_type=jax.ShapeDtypeStruct((NC, 2, NP), jnp.float32),
    mesh=_mesh,
    scratch_types=[
        pltpu.VMEM((TPW, CHUNK), jnp.int32),
        pltpu.VMEM((TPW, CHUNK), jnp.int32),
        pltpu.VMEM((CHUNK,), jnp.float32),
        pltpu.VMEM_SHARED((NP,), jnp.float32),
        pltpu.VMEM_SHARED((NP,), jnp.float32),
    ],
)
def _degree_kernel(src_hbm, dst_hbm, zeros_hbm, out_hbm,
                   src_v, dst_v, ones_v, dego_sh, degi_sh):
    cid = lax.axis_index("c")
    sid = lax.axis_index("s")
    wid = cid * NS + sid

    @pl.when(sid == 0)
    def _():
        pltpu.sync_copy(zeros_hbm, dego_sh)
        pltpu.sync_copy(zeros_hbm, degi_sh)

    pltpu.sync_copy(src_hbm.at[wid], src_v)
    pltpu.sync_copy(dst_hbm.at[wid], dst_v)
    for i in range(CHUNK // LANES):
        ones_v[pl.ds(i * LANES, LANES)] = jnp.ones((LANES,), jnp.float32)
    plsc.subcore_barrier()

    def body(j, carry):
        pltpu.sync_copy(ones_v, dego_sh.at[src_v.at[j]], add=True)
        pltpu.sync_copy(ones_v, degi_sh.at[dst_v.at[j]], add=True)
        return carry

    lax.fori_loop(0, TPW, body, 0)
    plsc.subcore_barrier()

    r0 = sid * ROWS_PT
    pltpu.sync_copy(dego_sh.at[pl.ds(r0, ROWS_PT)],
                    out_hbm.at[cid, 0, pl.ds(r0, ROWS_PT)])
    pltpu.sync_copy(degi_sh.at[pl.ds(r0, ROWS_PT)],
                    out_hbm.at[cid, 1, pl.ds(r0, ROWS_PT)])


@functools.partial(
    pl.kernel,
    out_type=jax.ShapeDtypeStruct((NC, NP, NHID), jnp.float32),
    mesh=_mesh,
    scratch_types=[
        pltpu.VMEM((HTPW, CHUNK), jnp.int32),
        pltpu.VMEM((HTPW, CHUNK), jnp.int32),
        pltpu.VMEM((CHUNK, NHID), jnp.float32),
        pltpu.VMEM((CHUNK, NHID), jnp.float32),
        pltpu.VMEM_SHARED((NP, NHID), jnp.float32),
        pltpu.SemaphoreType.DMA,
        pltpu.SemaphoreType.DMA,
    ],
)
def _scatter_kernel(h_hbm, src_hbm, dst_hbm, out_hbm,
                    src_v, dst_v, rows0_v, rows1_v, agg_sh, sem0, sem1):
    cid = lax.axis_index("c")
    sid = lax.axis_index("s")
    wid = cid * NS + sid
    r0 = sid * ROWS_PT

    def zbody(j, carry):
        for i in range(CHUNK // LANES):
            rows0_v[j, pl.ds(i * LANES, LANES)] = jnp.zeros((LANES,),
                                                            jnp.float32)
        return carry

    lax.fori_loop(0, CHUNK, zbody, 0)
    for k in range(ROWS_PT // CHUNK):
        pltpu.sync_copy(rows0_v, agg_sh.at[pl.ds(r0 + k * CHUNK, CHUNK)])
    plsc.subcore_barrier()

    for phase in range(2):
        pltpu.sync_copy(src_hbm.at[wid, pl.ds(phase * HTPW, HTPW)], src_v)
        pltpu.sync_copy(dst_hbm.at[wid, pl.ds(phase * HTPW, HTPW)], dst_v)
        pltpu.async_copy(h_hbm.at[src_v.at[0]], rows0_v, sem0)

        def body(g, carry):
            j0 = 2 * g
            pltpu.make_async_copy(h_hbm.at[src_v.at[j0]], rows0_v, sem0).wait()
            pltpu.async_copy(h_hbm.at[src_v.at[j0 + 1]], rows1_v, sem1)
            pltpu.sync_copy(rows0_v, agg_sh.at[dst_v.at[j0]], add=True)
            pltpu.make_async_copy(h_hbm.at[src_v.at[j0 + 1]], rows1_v,
                                  sem1).wait()
            pltpu.async_copy(h_hbm.at[src_v.at[j0 + 2]], rows0_v, sem0)
            pltpu.sync_copy(rows1_v, agg_sh.at[dst_v.at[j0 + 1]], add=True)
            return carry

        lax.fori_loop(0, HTPW // 2 - 1, body, 0)
        pltpu.make_async_copy(h_hbm.at[src_v.at[HTPW - 2]], rows0_v,
                              sem0).wait()
        pltpu.async_copy(h_hbm.at[src_v.at[HTPW - 1]], rows1_v, sem1)
        pltpu.sync_copy(rows0_v, agg_sh.at[dst_v.at[HTPW - 2]], add=True)
        pltpu.make_async_copy(h_hbm.at[src_v.at[HTPW - 1]], rows1_v,
                              sem1).wait()
        pltpu.sync_copy(rows1_v, agg_sh.at[dst_v.at[HTPW - 1]], add=True)
    plsc.subcore_barrier()

    pltpu.sync_copy(agg_sh.at[pl.ds(r0, ROWS_PT)],
                    out_hbm.at[cid, pl.ds(r0, ROWS_PT)])


BLK = 1024


def _mm1_body(degp_ref, x_ref, w_ref, h_ref):
    dp = degp_ref[...]
    deg = dp[0, 0, :] + dp[1, 0, :]
    norm = jnp.where(deg > 0, lax.rsqrt(jnp.maximum(deg, 1e-12)), 0.0)
    h_ref[...] = jnp.dot(x_ref[...] * norm[:, None], w_ref[...],
                         preferred_element_type=jnp.float32)


_h_call = pl.pallas_call(
    _mm1_body,
    grid=(NP // BLK,),
    in_specs=[
        pl.BlockSpec((NC, 2, BLK), lambda i: (0, 0, i)),
        pl.BlockSpec((BLK, NFEAT), lambda i: (i, 0)),
        pl.BlockSpec((NFEAT, NHID), lambda i: (0, 0)),
    ],
    out_specs=pl.BlockSpec((BLK, NHID), lambda i: (i, 0)),
    out_shape=jax.ShapeDtypeStruct((NP, NHID), jnp.float32),
)


def _mm2_body(aggp_ref, degp_ref, b1_ref, fcw_ref, fcb_ref, out_ref):
    ap = aggp_ref[...]
    agg = ap[0] + ap[1]
    dp = degp_ref[...]
    deg = dp[0, 1, :] + dp[1, 1, :]
    norm = jnp.where(deg > 0, lax.rsqrt(jnp.maximum(deg, 1e-12)), 0.0)
    h = jnp.maximum(agg * norm[:, None] + b1_ref[...], 0.0)
    out_ref[...] = (jnp.dot(h, fcw_ref[...], preferred_element_type=jnp.float32)
                    + fcb_ref[...])


_fc_call = pl.pallas_call(
    _mm2_body,
    grid=(NP // BLK,),
    in_specs=[
        pl.BlockSpec((NC, BLK, NHID), lambda i: (0, i, 0)),
        pl.BlockSpec((NC, 2, BLK), lambda i: (0, 0, i)),
        pl.BlockSpec((1, NHID), lambda i: (0, 0)),
        pl.BlockSpec((NHID, NCLASS), lambda i: (0, 0)),
        pl.BlockSpec((1, NCLASS), lambda i: (0, 0)),
    ],
    out_specs=pl.BlockSpec((BLK, NCLASS), lambda i: (i, 0)),
    out_shape=jax.ShapeDtypeStruct((NP, NCLASS), jnp.float32),
)


def kernel(edge_index, x, W1, b1, fc_W, fc_b):
    src = edge_index[0]
    dst = edge_index[1]
    pad = EP - E
    fill = N_NODES + (jnp.arange(pad, dtype=jnp.int32) % (NP - N_NODES))
    src_r = jnp.concatenate([src, fill]).reshape(NW, TPW, CHUNK)
    dst_r = jnp.concatenate([dst, fill]).reshape(NW, TPW, CHUNK)
    xp = jnp.zeros((NP, NFEAT), x.dtype).at[:N_NODES].set(x)
    zeros1 = jnp.zeros((NP,), jnp.float32)

    degp = _degree_kernel(src_r, dst_r, zeros1)
    h = _h_call(degp, xp, W1)
    aggp = _scatter_kernel(h, src_r, dst_r)
    out = _fc_call(aggp, degp, b1.reshape(1, NHID), fc_W,
                   fc_b.reshape(1, NCLASS))
    return out[:N_NODES]


# speedup vs baseline: 1.0273x; 1.0273x over previous
import functools

import jax
import jax.numpy as jnp
from jax import lax
from jax.experimental import pallas as pl
from jax.experimental.pallas import tpu as pltpu
from jax.experimental.pallas import tpu_sc as plsc

N_NODES = 10000
NP = 10240
E = 320000
NC, NS, LANES = 2, 16, 16
NW = NC * NS
CHUNK = 128
TPW = 80
HTPW = TPW // 2
EP = NW * TPW * CHUNK
NFEAT = 128
NHID = 128
NCLASS = 16
ROWS_PT = NP // NS

_mesh = plsc.VectorSubcoreMesh(
    core_axis_name="c", subcore_axis_name="s", num_cores=NC, num_subcores=NS)


@functools.partial(
    pl.kernel,
    out_type=jax.ShapeDtypeStruct((NC, 2, NP), jnp.float32),
    mesh=_mesh,
    scratch_types=[
        pltpu.VMEM((TPW, CHUNK), jnp.int32),
        pltpu.VMEM((TPW, CHUNK), jnp.int32),
        pltpu.VMEM((CHUNK,), jnp.float32),
        pltpu.VMEM_SHARED((NP,), jnp.float32),
        pltpu.VMEM_SHARED((NP,), jnp.float32),
        pltpu.SemaphoreType.DMA,
        pltpu.SemaphoreType.DMA,
    ],
)
def _degree_kernel(src_hbm, dst_hbm, zeros_hbm, out_hbm,
                   src_v, dst_v, ones_v, dego_sh, degi_sh, dsem0, dsem1):
    cid = lax.axis_index("c")
    sid = lax.axis_index("s")
    wid = cid * NS + sid

    @pl.when(sid == 0)
    def _():
        pltpu.sync_copy(zeros_hbm, dego_sh)
        pltpu.sync_copy(zeros_hbm, degi_sh)

    pltpu.sync_copy(src_hbm.at[wid], src_v)
    pltpu.sync_copy(dst_hbm.at[wid], dst_v)
    for i in range(CHUNK // LANES):
        ones_v[pl.ds(i * LANES, LANES)] = jnp.ones((LANES,), jnp.float32)
    plsc.subcore_barrier()

    def body(j, carry):
        pltpu.async_copy(ones_v, dego_sh.at[src_v.at[j]], dsem0, add=True)
        pltpu.async_copy(ones_v, degi_sh.at[dst_v.at[j]], dsem1, add=True)
        pltpu.make_async_copy(ones_v, dego_sh.at[src_v.at[j]], dsem0).wait()
        pltpu.make_async_copy(ones_v, degi_sh.at[dst_v.at[j]], dsem1).wait()
        return carry

    lax.fori_loop(0, TPW, body, 0)
    plsc.subcore_barrier()

    r0 = sid * ROWS_PT
    pltpu.sync_copy(dego_sh.at[pl.ds(r0, ROWS_PT)],
                    out_hbm.at[cid, 0, pl.ds(r0, ROWS_PT)])
    pltpu.sync_copy(degi_sh.at[pl.ds(r0, ROWS_PT)],
                    out_hbm.at[cid, 1, pl.ds(r0, ROWS_PT)])


@functools.partial(
    pl.kernel,
    out_type=jax.ShapeDtypeStruct((NC, NP, NHID), jnp.float32),
    mesh=_mesh,
    scratch_types=[
        pltpu.VMEM((HTPW, CHUNK), jnp.int32),
        pltpu.VMEM((HTPW, CHUNK), jnp.int32),
        pltpu.VMEM((CHUNK, NHID), jnp.float32),
        pltpu.VMEM((CHUNK, NHID), jnp.float32),
        pltpu.VMEM_SHARED((NP, NHID), jnp.float32),
        pltpu.SemaphoreType.DMA,
        pltpu.SemaphoreType.DMA,
    ],
)
def _scatter_kernel(h_hbm, src_hbm, dst_hbm, out_hbm,
                    src_v, dst_v, rows0_v, rows1_v, agg_sh, sem0, sem1):
    cid = lax.axis_index("c")
    sid = lax.axis_index("s")
    wid = cid * NS + sid
    r0 = sid * ROWS_PT

    def zbody(j, carry):
        for i in range(CHUNK // LANES):
            rows0_v[j, pl.ds(i * LANES, LANES)] = jnp.zeros((LANES,),
                                                            jnp.float32)
        return carry

    lax.fori_loop(0, CHUNK, zbody, 0)
    for k in range(ROWS_PT // CHUNK):
        pltpu.sync_copy(rows0_v, agg_sh.at[pl.ds(r0 + k * CHUNK, CHUNK)])
    plsc.subcore_barrier()

    for phase in range(2):
        pltpu.sync_copy(src_hbm.at[wid, pl.ds(phase * HTPW, HTPW)], src_v)
        pltpu.sync_copy(dst_hbm.at[wid, pl.ds(phase * HTPW, HTPW)], dst_v)
        pltpu.async_copy(h_hbm.at[src_v.at[0]], rows0_v, sem0)

        def body(g, carry):
            j0 = 2 * g
            pltpu.make_async_copy(h_hbm.at[src_v.at[j0]], rows0_v, sem0).wait()
            pltpu.async_copy(h_hbm.at[src_v.at[j0 + 1]], rows1_v, sem1)
            pltpu.sync_copy(rows0_v, agg_sh.at[dst_v.at[j0]], add=True)
            pltpu.make_async_copy(h_hbm.at[src_v.at[j0 + 1]], rows1_v,
                                  sem1).wait()
            pltpu.async_copy(h_hbm.at[src_v.at[j0 + 2]], rows0_v, sem0)
            pltpu.sync_copy(rows1_v, agg_sh.at[dst_v.at[j0 + 1]], add=True)
            return carry

        lax.fori_loop(0, HTPW // 2 - 1, body, 0)
        pltpu.make_async_copy(h_hbm.at[src_v.at[HTPW - 2]], rows0_v,
                              sem0).wait()
        pltpu.async_copy(h_hbm.at[src_v.at[HTPW - 1]], rows1_v, sem1)
        pltpu.sync_copy(rows0_v, agg_sh.at[dst_v.at[HTPW - 2]], add=True)
        pltpu.make_async_copy(h_hbm.at[src_v.at[HTPW - 1]], rows1_v,
                              sem1).wait()
        pltpu.sync_copy(rows1_v, agg_sh.at[dst_v.at[HTPW - 1]], add=True)
    plsc.subcore_barrier()

    pltpu.sync_copy(agg_sh.at[pl.ds(r0, ROWS_PT)],
                    out_hbm.at[cid, pl.ds(r0, ROWS_PT)])


BLK = 1024


def _mm1_body(degp_ref, x_ref, w_ref, h_ref):
    dp = degp_ref[...]
    deg = dp[0, 0, :] + dp[1, 0, :]
    norm = jnp.where(deg > 0, lax.rsqrt(jnp.maximum(deg, 1e-12)), 0.0)
    h_ref[...] = jnp.dot(x_ref[...] * norm[:, None], w_ref[...],
                         preferred_element_type=jnp.float32)


_h_call = pl.pallas_call(
    _mm1_body,
    grid=(NP // BLK,),
    in_specs=[
        pl.BlockSpec((NC, 2, BLK), lambda i: (0, 0, i)),
        pl.BlockSpec((BLK, NFEAT), lambda i: (i, 0)),
        pl.BlockSpec((NFEAT, NHID), lambda i: (0, 0)),
    ],
    out_specs=pl.BlockSpec((BLK, NHID), lambda i: (i, 0)),
    out_shape=jax.ShapeDtypeStruct((NP, NHID), jnp.float32),
)


def _mm2_body(aggp_ref, degp_ref, b1_ref, fcw_ref, fcb_ref, out_ref):
    ap = aggp_ref[...]
    agg = ap[0] + ap[1]
    dp = degp_ref[...]
    deg = dp[0, 1, :] + dp[1, 1, :]
    norm = jnp.where(deg > 0, lax.rsqrt(jnp.maximum(deg, 1e-12)), 0.0)
    h = jnp.maximum(agg * norm[:, None] + b1_ref[...], 0.0)
    out_ref[...] = (jnp.dot(h, fcw_ref[...], preferred_element_type=jnp.float32)
                    + fcb_ref[...])


_fc_call = pl.pallas_call(
    _mm2_body,
    grid=(NP // BLK,),
    in_specs=[
        pl.BlockSpec((NC, BLK, NHID), lambda i: (0, i, 0)),
        pl.BlockSpec((NC, 2, BLK), lambda i: (0, 0, i)),
        pl.BlockSpec((1, NHID), lambda i: (0, 0)),
        pl.BlockSpec((NHID, NCLASS), lambda i: (0, 0)),
        pl.BlockSpec((1, NCLASS), lambda i: (0, 0)),
    ],
    out_specs=pl.BlockSpec((BLK, NCLASS), lambda i: (i, 0)),
    out_shape=jax.ShapeDtypeStruct((NP, NCLASS), jnp.float32),
)


def kernel(edge_index, x, W1, b1, fc_W, fc_b):
    src = edge_index[0]
    dst = edge_index[1]
    pad = EP - E
    fill = N_NODES + (jnp.arange(pad, dtype=jnp.int32) % (NP - N_NODES))
    src_r = jnp.concatenate([src, fill]).reshape(NW, TPW, CHUNK)
    dst_r = jnp.concatenate([dst, fill]).reshape(NW, TPW, CHUNK)
    xp = jnp.zeros((NP, NFEAT), x.dtype).at[:N_NODES].set(x)
    zeros1 = jnp.zeros((NP,), jnp.float32)

    degp = _degree_kernel(src_r, dst_r, zeros1)
    h = _h_call(degp, xp, W1)
    aggp = _scatter_kernel(h, src_r, dst_r)
    out = _fc_call(aggp, degp, b1.reshape(1, NHID), fc_W,
                   fc_b.reshape(1, NCLASS))
    return out[:N_NODES]
